# baseline (speedup 1.0000x reference)
import functools

import jax
import jax.numpy as jnp
from jax.experimental import pallas as pl
from jax.experimental.pallas import tpu as pltpu

_SIZES = (16.0, 32.0, 64.0)
_RWIN = (40, 48, 56)


def _detloss_kernel(pred_ref, gtb_ref, gtl_ref, out_ref,
                    biou_s, bx1_s, by1_s, bx2_s, by2_s, blab_s, ay1_s, ay2_s,
                    nb_s, cnt_s, *, B, A, H, W, M):
    b = pl.program_id(0)
    P = (A * H, W)
    zero = jnp.zeros(P, jnp.float32)

    ii = jax.lax.broadcasted_iota(jnp.int32, P, 0)
    iw_col = jax.lax.broadcasted_iota(jnp.int32, (1, W), 1).astype(jnp.float32)
    cxv = (iw_col + 0.5) * 4.0

    @pl.when(b == 0)
    def _init_const():
        out_ref[0] = 0.0
        out_ref[1] = 0.0
        out_ref[2] = 0.0
        out_ref[3] = 0.0
        cyp = ((ii & 127).astype(jnp.float32) + 0.5) * 4.0
        hap = jnp.where(ii < H, _SIZES[0] / 2,
                        jnp.where(ii < 2 * H, _SIZES[1] / 2, _SIZES[2] / 2))
        ay1_s[...] = cyp - hap
        ay2_s[...] = cyp + hap

    biou_s[...] = zero
    bx1_s[...] = zero
    by1_s[...] = zero
    bx2_s[...] = zero
    by2_s[...] = zero
    blab_s[...] = zero

    cxm = [cxv - s / 2 for s in _SIZES]
    cxp = [cxv + s / 2 for s in _SIZES]

    def iou_body(m, carry):
        gx1 = gtb_ref[b, m, 0]
        gy1 = gtb_ref[b, m, 1]
        gx2 = gtb_ref[b, m, 2]
        gy2 = gtb_ref[b, m, 3]
        glab = gtl_ref[b, m].astype(jnp.float32)
        wb = gx2 - gx1
        hb = gy2 - gy1
        area_g = wb * hb
        for a in range(A):
            s = _SIZES[a]
            ha = s / 2
            R = _RWIN[a]
            im = jnp.minimum(wb, s) * jnp.minimum(hb, s)
            reachable = im * 1.399 >= 0.399 * (area_g + s * s)

            @pl.when(reachable)
            def _upd(a=a, s=s, ha=ha, R=R):
                r0f = (gy1 - ha) * 0.25 - 1.5
                r0i = jnp.floor(r0f).astype(jnp.int32)
                r0 = (jnp.clip(r0i, 0, H - R) // 8) * 8
                ds = pl.ds(a * H + r0, R)
                ih = jnp.maximum(
                    jnp.minimum(ay2_s[ds, :], gy2) - jnp.maximum(ay1_s[ds, :], gy1),
                    0.0)
                iw = jnp.maximum(
                    jnp.minimum(cxp[a], gx2) - jnp.maximum(cxm[a], gx1), 0.0)
                inter = ih * iw
                d = ((s * s + area_g) - inter) + 1e-6
                iou = inter / d
                bw = biou_s[ds, :]
                upd = iou > bw
                biou_s[ds, :] = jnp.where(upd, iou, bw)
                for ref, val in ((bx1_s, gx1), (by1_s, gy1), (bx2_s, gx2),
                                 (by2_s, gy2), (blab_s, glab)):
                    ref[ds, :] = jnp.where(upd, val, ref[ds, :])
        return carry

    jax.lax.fori_loop(0, M, iou_body, 0)

    biou = biou_s[...]
    pos = biou >= 0.5
    neg = biou < 0.4
    posf = pos.astype(jnp.float32)

    x = pred_ref[0, :, 4, :, :].reshape(P)
    raw = jnp.maximum(x, 0.0) - x * posf + jnp.log1p(jnp.exp(-jnp.abs(x)))

    def vsum(v):
        return jnp.sum(v, axis=(0, 1), keepdims=True)

    np_i = vsum(pos.astype(jnp.int32))
    nn_i = vsum(neg.astype(jnp.int32))
    spr = vsum(raw * posf)

    bits = jax.lax.bitcast_convert_type(raw, jnp.int32)
    nb_s[...] = jnp.where(neg, bits, -1)
    k_i = jnp.minimum(nn_i, jnp.maximum(1, 3 * jnp.maximum(1, np_i)))

    lo = jnp.zeros((1, 1), jnp.int32)

    def radix_pass(lo, step, njt):
        nbp = nb_s[...]
        for j in range(1, njt + 1):
            lane = jnp.sum((nbp >= lo + (j * step)).astype(jnp.int32),
                           axis=0, keepdims=True)
            cnt_s[j - 1:j, :] = lane
        tot = jnp.sum(cnt_s[...], axis=1, keepdims=True)
        sat = (tot >= k_i) & (jax.lax.broadcasted_iota(
            jnp.int32, (16, 1), 0) < njt)
        n_sat = jnp.sum(sat.astype(jnp.int32), axis=0, keepdims=True)
        return lo + n_sat * step

    for p in range(7):
        lo = radix_pass(lo, 1 << (27 - 4 * p), 15)
    t_bits = radix_pass(lo, 1, 7)
    nbits = nb_s[...]
    sel_le = neg & (nbits <= t_bits)
    t_val = jnp.max(jnp.where(sel_le, raw, -1.0), axis=(0, 1), keepdims=True)
    sel_gt = nbits > t_bits
    sum_gt = vsum(jnp.where(sel_gt, raw, 0.0))
    cnt_gt = vsum(sel_gt.astype(jnp.int32))
    sum_topk = jnp.where(nn_i > 0,
                         sum_gt + (k_i - cnt_gt).astype(jnp.float32) * t_val, 0.0)

    denom = jnp.maximum(np_i + k_i, 1).astype(jnp.float32)
    lob = (spr + sum_topk) / denom

    c0 = pred_ref[0, :, 5, :, :].reshape(P)
    c1 = pred_ref[0, :, 6, :, :].reshape(P)
    c2 = pred_ref[0, :, 7, :, :].reshape(P)
    mx = jnp.maximum(c0, jnp.maximum(c1, c2))
    lse = mx + jnp.log(jnp.exp(c0 - mx) + jnp.exp(c1 - mx) + jnp.exp(c2 - mx))
    tgt = blab_s[...] - 1.0
    picked = jnp.where(tgt == 0.0, c0, jnp.where(tgt == 1.0, c1, c2))
    sce = jnp.sum((lse - picked) * posf)

    eps = 1e-6
    sp = jnp.where(ii < H, _SIZES[0], jnp.where(ii < 2 * H, _SIZES[1], _SIZES[2]))
    cyp_full = ((ii & 127).astype(jnp.float32) + 0.5) * 4.0
    bx1 = bx1_s[...]
    by1 = by1_s[...]
    gw = bx2_s[...] - bx1
    gh = by2_s[...] - by1
    gcx = bx1 + 0.5 * gw
    gcy = by1 + 0.5 * gh
    tx = (gcx - cxv) / sp
    ty = (gcy - cyp_full) / sp
    tw = jnp.log(jnp.maximum(gw, eps) / sp)
    th = jnp.log(jnp.maximum(gh, eps) / sp)

    def sl1(d):
        ad = jnp.abs(d)
        return jnp.where(ad < 1.0, 0.5 * d * d, ad - 0.5)

    l1 = (sl1(pred_ref[0, :, 0, :, :].reshape(P) - tx)
          + sl1(pred_ref[0, :, 1, :, :].reshape(P) - ty)
          + sl1(pred_ref[0, :, 2, :, :].reshape(P) - tw)
          + sl1(pred_ref[0, :, 3, :, :].reshape(P) - th))
    sl1s = jnp.sum(l1 * posf)

    has_pos = np_i > 0
    safe = jnp.maximum(np_i, 1).astype(jnp.float32)
    lcb = jnp.where(has_pos, sce / safe, 0.0)
    llb = jnp.where(has_pos, sl1s / (safe * 4.0), 0.0)

    new_lo = out_ref[0] + lob[0, 0]
    new_lc = out_ref[1] + lcb[0, 0]
    new_ll = out_ref[2] + llb[0, 0]
    out_ref[0] = new_lo
    out_ref[1] = new_lc
    out_ref[2] = new_ll

    @pl.when(b == B - 1)
    def _fin():
        flo = new_lo / B
        flc = new_lc / B
        fll = new_ll / B
        out_ref[0] = flo
        out_ref[1] = flc
        out_ref[2] = fll
        out_ref[3] = flo + flc + 2.0 * fll


@jax.jit
def kernel(pred, anchors, gt_boxes, gt_labels):
    B, ch, H, W = pred.shape
    A = anchors.shape[0] // (H * W)
    chpa = ch // A
    M = gt_boxes.shape[1]
    pred_r = pred.reshape(B, A, chpa, H, W)

    scratch = [pltpu.VMEM((A * H, W), jnp.float32) for _ in range(8)]
    scratch.append(pltpu.VMEM((A * H, W), jnp.int32))
    scratch.append(pltpu.VMEM((16, W), jnp.int32))
    out = pl.pallas_call(
        functools.partial(_detloss_kernel, B=B, A=A, H=H, W=W, M=M),
        grid=(B,),
        in_specs=[
            pl.BlockSpec((1, A, chpa, H, W), lambda b: (b, 0, 0, 0, 0)),
            pl.BlockSpec(memory_space=pltpu.SMEM),
            pl.BlockSpec(memory_space=pltpu.SMEM),
        ],
        out_specs=pl.BlockSpec(memory_space=pltpu.SMEM),
        out_shape=jax.ShapeDtypeStruct((4,), jnp.float32),
        scratch_shapes=scratch,
    )(pred_r, gt_boxes, gt_labels)
    return (out[0], out[1], out[2], out[3])


# speedup vs baseline: 1.0540x; 1.0540x over previous
import functools

import jax
import jax.numpy as jnp
from jax.experimental import pallas as pl
from jax.experimental.pallas import tpu as pltpu

_SIZES = (16.0, 32.0, 64.0)
_RWIN = (40, 48, 56)


def _detloss_kernel(pred_ref, gtb_ref, gtl_ref, out_ref,
                    biou_s, bx1_s, by1_s, bx2_s, by2_s, blab_s, ay1_s, ay2_s,
                    nb_s, *, B, A, H, W, M):
    b = pl.program_id(0)
    P = (A * H, W)
    zero = jnp.zeros(P, jnp.float32)

    ii = jax.lax.broadcasted_iota(jnp.int32, P, 0)
    iw_col = jax.lax.broadcasted_iota(jnp.int32, (1, W), 1).astype(jnp.float32)
    cxv = (iw_col + 0.5) * 4.0

    @pl.when(b == 0)
    def _init_const():
        out_ref[0] = 0.0
        out_ref[1] = 0.0
        out_ref[2] = 0.0
        out_ref[3] = 0.0
        cyp = ((ii & 127).astype(jnp.float32) + 0.5) * 4.0
        hap = jnp.where(ii < H, _SIZES[0] / 2,
                        jnp.where(ii < 2 * H, _SIZES[1] / 2, _SIZES[2] / 2))
        ay1_s[...] = cyp - hap
        ay2_s[...] = cyp + hap

    biou_s[...] = zero
    bx1_s[...] = zero
    by1_s[...] = zero
    bx2_s[...] = zero
    by2_s[...] = zero
    blab_s[...] = zero

    cxm = [cxv - s / 2 for s in _SIZES]
    cxp = [cxv + s / 2 for s in _SIZES]

    def iou_body(m, carry):
        gx1 = gtb_ref[b, m, 0]
        gy1 = gtb_ref[b, m, 1]
        gx2 = gtb_ref[b, m, 2]
        gy2 = gtb_ref[b, m, 3]
        glab = gtl_ref[b, m].astype(jnp.float32)
        wb = gx2 - gx1
        hb = gy2 - gy1
        area_g = wb * hb
        for a in range(A):
            s = _SIZES[a]
            ha = s / 2
            R = _RWIN[a]
            im = jnp.minimum(wb, s) * jnp.minimum(hb, s)
            reachable = im * 1.399 >= 0.399 * (area_g + s * s)

            @pl.when(reachable)
            def _upd(a=a, s=s, ha=ha, R=R):
                r0f = (gy1 - ha) * 0.25 - 1.5
                r0i = jnp.floor(r0f).astype(jnp.int32)
                r0 = (jnp.clip(r0i, 0, H - R) // 8) * 8
                ds = pl.ds(a * H + r0, R)
                ih = jnp.maximum(
                    jnp.minimum(ay2_s[ds, :], gy2) - jnp.maximum(ay1_s[ds, :], gy1),
                    0.0)
                iw = jnp.maximum(
                    jnp.minimum(cxp[a], gx2) - jnp.maximum(cxm[a], gx1), 0.0)
                inter = ih * iw
                d = ((s * s + area_g) - inter) + 1e-6
                iou = inter / d
                bw = biou_s[ds, :]
                upd = iou > bw
                biou_s[ds, :] = jnp.where(upd, iou, bw)
                for ref, val in ((bx1_s, gx1), (by1_s, gy1), (bx2_s, gx2),
                                 (by2_s, gy2), (blab_s, glab)):
                    ref[ds, :] = jnp.where(upd, val, ref[ds, :])
        return carry

    for m in range(M):
        iou_body(m, 0)

    biou = biou_s[...]
    pos = biou >= 0.5
    neg = biou < 0.4
    posf = pos.astype(jnp.float32)

    x = pred_ref[0, :, 4, :, :].reshape(P)
    raw = jnp.maximum(x, 0.0) - x * posf + jnp.log1p(jnp.exp(-jnp.abs(x)))

    def vsum(v):
        return jnp.sum(v, axis=(0, 1), keepdims=True)

    np_i = vsum(pos.astype(jnp.int32))
    nn_i = vsum(neg.astype(jnp.int32))
    spr = vsum(raw * posf)

    bits = jax.lax.bitcast_convert_type(raw, jnp.int32)
    nb_s[...] = jnp.where(neg, bits, -1)
    k_i = jnp.minimum(nn_i, jnp.maximum(1, 3 * jnp.maximum(1, np_i)))

    lo = jnp.zeros((1, 1), jnp.int32)

    def radix_pass(lo, step, njt):
        nbp = nb_s[...]
        n_sat = jnp.zeros((1, 1), jnp.int32)
        for j in range(1, njt + 1):
            cnt = vsum((nbp >= lo + (j * step)).astype(jnp.int32))
            n_sat = n_sat + (cnt >= k_i).astype(jnp.int32)
        return lo + n_sat * step

    for p in range(7):
        lo = radix_pass(lo, 1 << (27 - 4 * p), 15)
    t_bits = radix_pass(lo, 1, 7)
    nbits = nb_s[...]
    sel_le = neg & (nbits <= t_bits)
    t_val = jnp.max(jnp.where(sel_le, raw, -1.0), axis=(0, 1), keepdims=True)
    sel_gt = nbits > t_bits
    sum_gt = vsum(jnp.where(sel_gt, raw, 0.0))
    cnt_gt = vsum(sel_gt.astype(jnp.int32))
    sum_topk = jnp.where(nn_i > 0,
                         sum_gt + (k_i - cnt_gt).astype(jnp.float32) * t_val, 0.0)

    denom = jnp.maximum(np_i + k_i, 1).astype(jnp.float32)
    lob = (spr + sum_topk) / denom

    c0 = pred_ref[0, :, 5, :, :].reshape(P)
    c1 = pred_ref[0, :, 6, :, :].reshape(P)
    c2 = pred_ref[0, :, 7, :, :].reshape(P)
    mx = jnp.maximum(c0, jnp.maximum(c1, c2))
    lse = mx + jnp.log(jnp.exp(c0 - mx) + jnp.exp(c1 - mx) + jnp.exp(c2 - mx))
    tgt = blab_s[...] - 1.0
    picked = jnp.where(tgt == 0.0, c0, jnp.where(tgt == 1.0, c1, c2))
    sce = jnp.sum((lse - picked) * posf)

    eps = 1e-6
    sp = jnp.where(ii < H, _SIZES[0], jnp.where(ii < 2 * H, _SIZES[1], _SIZES[2]))
    cyp_full = ((ii & 127).astype(jnp.float32) + 0.5) * 4.0
    bx1 = bx1_s[...]
    by1 = by1_s[...]
    gw = bx2_s[...] - bx1
    gh = by2_s[...] - by1
    gcx = bx1 + 0.5 * gw
    gcy = by1 + 0.5 * gh
    tx = (gcx - cxv) / sp
    ty = (gcy - cyp_full) / sp
    tw = jnp.log(jnp.maximum(gw, eps) / sp)
    th = jnp.log(jnp.maximum(gh, eps) / sp)

    def sl1(d):
        ad = jnp.abs(d)
        return jnp.where(ad < 1.0, 0.5 * d * d, ad - 0.5)

    l1 = (sl1(pred_ref[0, :, 0, :, :].reshape(P) - tx)
          + sl1(pred_ref[0, :, 1, :, :].reshape(P) - ty)
          + sl1(pred_ref[0, :, 2, :, :].reshape(P) - tw)
          + sl1(pred_ref[0, :, 3, :, :].reshape(P) - th))
    sl1s = jnp.sum(l1 * posf)

    has_pos = np_i > 0
    safe = jnp.maximum(np_i, 1).astype(jnp.float32)
    lcb = jnp.where(has_pos, sce / safe, 0.0)
    llb = jnp.where(has_pos, sl1s / (safe * 4.0), 0.0)

    new_lo = out_ref[0] + lob[0, 0]
    new_lc = out_ref[1] + lcb[0, 0]
    new_ll = out_ref[2] + llb[0, 0]
    out_ref[0] = new_lo
    out_ref[1] = new_lc
    out_ref[2] = new_ll

    @pl.when(b == B - 1)
    def _fin():
        flo = new_lo / B
        flc = new_lc / B
        fll = new_ll / B
        out_ref[0] = flo
        out_ref[1] = flc
        out_ref[2] = fll
        out_ref[3] = flo + flc + 2.0 * fll


@jax.jit
def kernel(pred, anchors, gt_boxes, gt_labels):
    B, ch, H, W = pred.shape
    A = anchors.shape[0] // (H * W)
    chpa = ch // A
    M = gt_boxes.shape[1]
    pred_r = pred.reshape(B, A, chpa, H, W)

    scratch = [pltpu.VMEM((A * H, W), jnp.float32) for _ in range(8)]
    scratch.append(pltpu.VMEM((A * H, W), jnp.int32))
    out = pl.pallas_call(
        functools.partial(_detloss_kernel, B=B, A=A, H=H, W=W, M=M),
        grid=(B,),
        in_specs=[
            pl.BlockSpec((1, A, chpa, H, W), lambda b: (b, 0, 0, 0, 0)),
            pl.BlockSpec(memory_space=pltpu.SMEM),
            pl.BlockSpec(memory_space=pltpu.SMEM),
        ],
        out_specs=pl.BlockSpec(memory_space=pltpu.SMEM),
        out_shape=jax.ShapeDtypeStruct((4,), jnp.float32),
        scratch_shapes=scratch,
    )(pred_r, gt_boxes, gt_labels)
    return (out[0], out[1], out[2], out[3])
